# baseline (speedup 1.0000x reference)
import functools

import jax
import jax.numpy as jnp
import numpy as np
from jax import lax
from jax.experimental import pallas as pl
from jax.experimental.pallas import tpu as pltpu
from jax.experimental.pallas import tpu_sc as plsc

B = 1024
D = 256
H = 8
DH = 32
NL = 4
T = 72
TREAL = 65
BS = 32

GT = 64

O_IB = 224
O_SE = 288
O_Z = 1170
T128_ROWS = 1176
SE_INNER = (0, 61, 162, 263, 303, 431)

SM_ROWS = 384
SM_OFF = (0, 64, 128, 192)
SHOP_ROWS = 384

_NW = 32
_GROWS = B * GT
_RPW = _GROWS // _NW
_CHUNK = 256
_NCH = _RPW // _CHUNK
_NBUF = 3


def _sc_gather(table, idx):
    mesh = plsc.VectorSubcoreMesh(core_axis_name="c", subcore_axis_name="s")

    @functools.partial(
        pl.kernel, mesh=mesh,
        out_type=jax.ShapeDtypeStruct((_GROWS, 128), jnp.float32),
        scratch_types=[
            pltpu.VMEM((_RPW,), jnp.int32),
        ] + [pltpu.VMEM((_CHUNK, 128), jnp.float32)] * _NBUF + [
            pltpu.SemaphoreType.DMA,
            pltpu.SemaphoreType.DMA,
        ],
    )
    def gather_k(table_hbm, idx_hbm, out_hbm, idx_v, r0, r1, r2, sem_g, sem_s):
        bufs = (r0, r1, r2)
        wid = lax.axis_index("s") * 2 + lax.axis_index("c")
        base = wid * _RPW
        pltpu.sync_copy(idx_hbm.at[pl.ds(base, _RPW)], idx_v)

        def fire(c):
            return pltpu.async_copy(
                table_hbm.at[idx_v.at[pl.ds(c * _CHUNK, _CHUNK)]],
                bufs[c % _NBUF], sem_g)

        gets = {c: fire(c) for c in range(_NBUF)}
        puts = {}
        for c in range(_NCH):
            gets[c].wait()
            puts[c] = pltpu.async_copy(
                bufs[c % _NBUF], out_hbm.at[pl.ds(base + c * _CHUNK, _CHUNK)],
                sem_s)
            if c + _NBUF < _NCH:
                puts[c].wait()
                gets[c + _NBUF] = fire(c + _NBUF)
        for c in range(max(0, _NCH - _NBUF), _NCH):
            puts[c].wait()

    return gather_k(table, idx)


def _multihot_gather(idx, tab, nrows):
    rows = idx.shape[0]
    iota = lax.broadcasted_iota(jnp.int32, (rows, nrows), 1)
    mh = jnp.zeros((rows, nrows), jnp.float32)
    for f in range(idx.shape[1]):
        mh = mh + (iota == idx[:, f:f + 1]).astype(jnp.float32)
    return jnp.dot(mh, tab, preferred_element_type=jnp.float32)


def _ln(xf, g, b):
    m = jnp.mean(xf, axis=-1, keepdims=True)
    v = jnp.mean((xf - m) * (xf - m), axis=-1, keepdims=True)
    return (xf - m) * lax.rsqrt(v + 1e-5) * g + b


def _fused_kernel(gath_ref, usm_idx_ref, shop_idx_ref,
                  traits_ref, scalars_ref, sm_tab_ref, shop_tab_ref,
                  tw1_ref, tb1_ref, tw2_ref, tb2_ref, tw3_ref, tb3_ref,
                  sw1_ref, sb1_ref, sw2_ref, sb2_ref, sw3_ref, sb3_ref,
                  base_ref,
                  wq_ref, bq_ref, wk_ref, bk_ref, wv_ref, bv_ref,
                  wo_ref, bo_ref, ln1g_ref, ln1b_ref,
                  w1_ref, b1_ref, w2_ref, b2_ref, ln2g_ref, ln2b_ref,
                  fw1_ref, fb1_ref, fw2_ref, fb2_ref, fw3_ref, fb3_ref,
                  out_ref, x_ref, o_ref):
    f32 = jnp.float32

    g2 = gath_ref[...].reshape(BS, GT, 128)
    usm = _multihot_gather(usm_idx_ref[...], sm_tab_ref[...], SM_ROWS)
    usm = usm.reshape(BS, 40, 128)
    sh = _multihot_gather(shop_idx_ref[...], shop_tab_ref[...], SHOP_ROWS)
    sh = sh.reshape(BS, 8, D)

    t = traits_ref[...]
    t = jnp.maximum(jnp.dot(t, tw1_ref[...], preferred_element_type=f32) + tb1_ref[...], 0.0)
    t = jnp.maximum(jnp.dot(t, tw2_ref[...], preferred_element_type=f32) + tb2_ref[...], 0.0)
    t = jnp.dot(t, tw3_ref[...], preferred_element_type=f32) + tb3_ref[...]
    t = t.reshape(BS, 8, D)

    s = scalars_ref[...]
    s = jnp.maximum(jnp.dot(s, sw1_ref[...], preferred_element_type=f32) + sb1_ref[...], 0.0)
    s = jnp.maximum(jnp.dot(s, sw2_ref[...], preferred_element_type=f32) + sb2_ref[...], 0.0)
    s = jnp.dot(s, sw3_ref[...], preferred_element_type=f32) + sb3_ref[...]
    s = s.reshape(BS, 8, D)

    x_ref[:, 0:4, :] = jnp.zeros((BS, 4, D), f32)
    x_ref[:, 4:32, 0:128] = g2[:, 0:28]
    x_ref[:, 4:32, 128:256] = usm[:, 0:28]
    x_ref[:, 32:39, :] = t[:, 0:7]
    x_ref[:, 39:44, 0:128] = g2[:, 37:42]
    x_ref[:, 39:44, 128:256] = g2[:, 42:47]
    x_ref[:, 44:53, 0:128] = g2[:, 28:37]
    x_ref[:, 44:53, 128:256] = usm[:, 28:37]
    x_ref[:, 53:58, :] = sh[:, 0:5]
    x_ref[:, 58:59, :] = s[:, 0:1]
    x_ref[:, 59:65, 0:128] = g2[:, 47:53]
    x_ref[:, 59:65, 128:256] = g2[:, 53:59]
    x_ref[:, 65:72, :] = jnp.zeros((BS, 7, D), f32)

    xf = (x_ref[...] + base_ref[...][None]).reshape(BS * T, D)

    kmask = jnp.where(
        lax.broadcasted_iota(jnp.int32, (1, 1, T), 2) >= TREAL, -1e30, 0.0)

    for l in range(NL):
        q = jnp.dot(xf, wq_ref[l], preferred_element_type=f32) + bq_ref[l:l + 1]
        k = jnp.dot(xf, wk_ref[l], preferred_element_type=f32) + bk_ref[l:l + 1]
        v = jnp.dot(xf, wv_ref[l], preferred_element_type=f32) + bv_ref[l:l + 1]
        q3 = q.reshape(BS, T, D)
        k3 = k.reshape(BS, T, D)
        v3 = v.reshape(BS, T, D)
        for h in range(H):
            qh = q3[:, :, h * DH:(h + 1) * DH]
            kh = k3[:, :, h * DH:(h + 1) * DH]
            vh = v3[:, :, h * DH:(h + 1) * DH]
            e = jnp.exp(lax.dot_general(
                qh, kh, (((2,), (2,)), ((0,), (0,))),
                preferred_element_type=f32) + kmask)
            rs = lax.reciprocal(jnp.sum(e, axis=-1, keepdims=True))
            o_ref[:, :, h * DH:(h + 1) * DH] = lax.dot_general(
                e, vh, (((2,), (1,)), ((0,), (0,))),
                preferred_element_type=f32) * rs
        o = o_ref[...].reshape(BS * T, D)
        attn = jnp.dot(o, wo_ref[l], preferred_element_type=f32) + bo_ref[l:l + 1]
        xf = _ln(xf + attn, ln1g_ref[l:l + 1], ln1b_ref[l:l + 1])
        h1 = jnp.maximum(jnp.dot(xf, w1_ref[l], preferred_element_type=f32) + b1_ref[l:l + 1], 0.0)
        h2 = jnp.dot(h1, w2_ref[l], preferred_element_type=f32) + b2_ref[l:l + 1]
        xf = _ln(xf + h2, ln2g_ref[l:l + 1], ln2b_ref[l:l + 1])

    x3 = xf.reshape(BS, T, D)
    acc = jnp.zeros((BS, D), f32)
    for tt in range(4):
        acc = acc + jnp.dot(x3[:, tt, :], fw1_ref[tt], preferred_element_type=f32)
    h1 = jnp.maximum(acc + fb1_ref[...], 0.0)
    h2 = jnp.maximum(jnp.dot(h1, fw2_ref[...], preferred_element_type=f32) + fb2_ref[...], 0.0)
    out_ref[...] = jnp.dot(h2, fw3_ref[...], preferred_element_type=f32) + fb3_ref[...]


def kernel(board, bench, shop, items, traits, scalars, emb_scalars, params):
    p = params
    f32 = jnp.float32
    i32 = jnp.int32
    bs = board.shape[0]

    se_stack = jnp.concatenate([
        p['gold_emb'], p['health_emb'], p['exp_emb'],
        p['round_emb'], p['oppo_emb'], p['level_emb']], axis=0)
    t128 = jnp.concatenate([
        p['champion_emb'],
        jnp.zeros((O_IB - 221, 128), f32),
        p['item_bench_emb'],
        jnp.zeros((O_SE - O_IB - 58, 128), f32),
        se_stack.reshape(882, 128),
        jnp.zeros((T128_ROWS - O_Z, 128), f32),
    ], axis=0)

    units = jnp.concatenate(
        [board.reshape(bs, 28, 5), bench.reshape(bs, 9, 5)], axis=1).astype(i32)
    items = items.astype(i32)
    se_v = emb_scalars.astype(i32) + jnp.array(SE_INNER, i32)[None, :]
    gidx = jnp.concatenate([
        units[:, :, 0],
        O_IB + items[:, 0::2],
        O_IB + items[:, 1::2],
        O_SE + 2 * se_v,
        O_SE + 2 * se_v + 1,
        jnp.full((bs, 5), O_Z, i32),
    ], axis=1).reshape(bs * GT)

    gath = _sc_gather(t128, gidx)

    sm_tab = jnp.zeros((SM_ROWS, 128), f32)
    sm_tab = sm_tab.at[0:58, 0:32].set(p['item_emb_1'])
    sm_tab = sm_tab.at[64:122, 32:64].set(p['item_emb_2'])
    sm_tab = sm_tab.at[128:186, 64:96].set(p['item_emb_3'])
    sm_tab = sm_tab.at[192:337, 96:128].set(p['champ_trait_emb'])

    shop_tab = jnp.zeros((SHOP_ROWS, D), f32)
    shop_tab = shop_tab.at[0:221, 0:192].set(p['shop_champ_emb'])
    shop_tab = shop_tab.at[224:369, 192:256].set(p['shop_trait_emb'])

    usm_idx = units[:, :, 1:5] + jnp.array(SM_OFF, i32)[None, None, :]
    usm_idx = jnp.pad(usm_idx, ((0, 0), (0, 3), (0, 0))).reshape(bs * 40, 4)

    shop = shop.astype(i32)
    shop_idx = jnp.stack([shop[..., 0], shop[..., 4] + 224], axis=-1)
    shop_idx = jnp.pad(shop_idx, ((0, 0), (0, 3), (0, 0))).reshape(bs * 8, 2)

    traits_p = jnp.pad(traits, ((0, 0), (0, 1), (0, 26))).reshape(bs * 8, 128)
    scalars_p = jnp.pad(scalars, ((0, 0), (0, 7), (0, 4))).reshape(bs * 8, 32)

    tm = p['trait_mlp']
    tw1 = jnp.pad(tm[0][0], ((0, 26), (0, 0)))
    tb1, tw2, tb2, tw3, tb3 = tm[0][1][None], tm[1][0], tm[1][1][None], tm[2][0], tm[2][1][None]
    sm = p['scalar_mlp']
    sw1 = jnp.pad(sm[0][0], ((0, 4), (0, 0)))
    sb1, sw2, sb2, sw3, sb3 = sm[0][1][None], sm[1][0], sm[1][1][None], sm[2][0], sm[2][1][None]
    fp = p['feature_proc']
    fw1 = fp[0][0].reshape(4, D, D)
    fb1, fw2, fb2, fw3, fb3 = fp[0][1][None], fp[1][0], fp[1][1][None], fp[2][0], fp[2][1][None]

    base = jnp.zeros((T, D), f32)
    base = base.at[0:4, :].set(p['cls_token'][0])
    base = base.at[4:65, :].set(p['pos_emb'][0:61])

    L = p['layers']

    def stk(name):
        return jnp.stack([L[l][name] for l in range(NL)])

    iscale = 1.0 / np.sqrt(DH)
    wq, bq, wk, bk = stk('Wq') * iscale, stk('bq') * iscale, stk('Wk'), stk('bk')
    wv, bv, wo, bo = stk('Wv'), stk('bv'), stk('Wo'), stk('bo')
    ln1g, ln1b = stk('ln1_g'), stk('ln1_b')
    w1, b1, w2, b2 = stk('W1'), stk('b1'), stk('W2'), stk('b2')
    ln2g, ln2b = stk('ln2_g'), stk('ln2_b')

    def bspec(shape, blocked_rows=None):
        if blocked_rows is None:
            nd = len(shape)
            return pl.BlockSpec(shape, lambda i: (0,) * nd)
        return pl.BlockSpec((blocked_rows,) + shape[1:],
                            lambda i: (i,) + (0,) * (len(shape) - 1))

    ins = [
        (gath, BS * GT), (usm_idx, BS * 40), (shop_idx, BS * 8),
        (traits_p, BS * 8), (scalars_p, BS * 8),
        (sm_tab, None), (shop_tab, None),
        (tw1, None), (tb1, None), (tw2, None), (tb2, None), (tw3, None), (tb3, None),
        (sw1, None), (sb1, None), (sw2, None), (sb2, None), (sw3, None), (sb3, None),
        (base, None),
        (wq, None), (bq, None), (wk, None), (bk, None), (wv, None), (bv, None),
        (wo, None), (bo, None), (ln1g, None), (ln1b, None),
        (w1, None), (b1, None), (w2, None), (b2, None), (ln2g, None), (ln2b, None),
        (fw1, None), (fb1, None), (fw2, None), (fb2, None), (fw3, None), (fb3, None),
    ]

    out = pl.pallas_call(
        _fused_kernel,
        grid=(bs // BS,),
        in_specs=[bspec(a.shape, r) for a, r in ins],
        out_specs=pl.BlockSpec((BS, 1024), lambda i: (i, 0)),
        out_shape=jax.ShapeDtypeStruct((bs, 1024), f32),
        scratch_shapes=[pltpu.VMEM((BS, T, D), f32), pltpu.VMEM((BS, T, D), f32)],
        compiler_params=pltpu.CompilerParams(
            dimension_semantics=("arbitrary",)),
    )(*[a for a, _ in ins])
    return out


# speedup vs baseline: 1.0059x; 1.0059x over previous
import functools

import jax
import jax.numpy as jnp
import numpy as np
from jax import lax
from jax.experimental import pallas as pl
from jax.experimental.pallas import tpu as pltpu
from jax.experimental.pallas import tpu_sc as plsc

B = 1024
D = 256
H = 8
DH = 32
NL = 4
T = 72
TREAL = 65
BS = 32

GT = 64

O_IB = 224
O_SE = 288
O_Z = 1170
T128_ROWS = 1176
SE_INNER = (0, 61, 162, 263, 303, 431)

SM_ROWS = 384
SM_OFF = (0, 64, 128, 192)
SHOP_ROWS = 384

_NW = 32
NSPLIT = 4
BSL = B // NSPLIT
_GROWS = BSL * GT
_RPW = _GROWS // _NW
_CHUNK = 256
_NCH = _RPW // _CHUNK
_NBUF = min(3, _NCH)


def _sc_gather(table, idx):
    mesh = plsc.VectorSubcoreMesh(core_axis_name="c", subcore_axis_name="s")

    @functools.partial(
        pl.kernel, mesh=mesh,
        out_type=jax.ShapeDtypeStruct((_GROWS, 128), jnp.float32),
        scratch_types=[
            pltpu.VMEM((_RPW,), jnp.int32),
        ] + [pltpu.VMEM((_CHUNK, 128), jnp.float32)] * _NBUF + [
            pltpu.SemaphoreType.DMA,
            pltpu.SemaphoreType.DMA,
        ],
    )
    def gather_k(table_hbm, idx_hbm, out_hbm, idx_v, *rest):
        bufs = rest[:_NBUF]
        sem_g, sem_s = rest[_NBUF], rest[_NBUF + 1]
        wid = lax.axis_index("s") * 2 + lax.axis_index("c")
        base = wid * _RPW
        pltpu.sync_copy(idx_hbm.at[pl.ds(base, _RPW)], idx_v)

        def fire(c):
            return pltpu.async_copy(
                table_hbm.at[idx_v.at[pl.ds(c * _CHUNK, _CHUNK)]],
                bufs[c % _NBUF], sem_g)

        gets = {c: fire(c) for c in range(_NBUF)}
        puts = {}
        for c in range(_NCH):
            gets[c].wait()
            puts[c] = pltpu.async_copy(
                bufs[c % _NBUF], out_hbm.at[pl.ds(base + c * _CHUNK, _CHUNK)],
                sem_s)
            if c + _NBUF < _NCH:
                puts[c].wait()
                gets[c + _NBUF] = fire(c + _NBUF)
        for c in range(max(0, _NCH - _NBUF), _NCH):
            puts[c].wait()

    return gather_k(table, idx)


def _multihot_gather(idx, tab, nrows):
    rows = idx.shape[0]
    iota = lax.broadcasted_iota(jnp.int32, (rows, nrows), 1)
    mh = jnp.zeros((rows, nrows), jnp.float32)
    for f in range(idx.shape[1]):
        mh = mh + (iota == idx[:, f:f + 1]).astype(jnp.float32)
    return jnp.dot(mh, tab, preferred_element_type=jnp.float32)


def _ln(xf, g, b):
    m = jnp.mean(xf, axis=-1, keepdims=True)
    v = jnp.mean((xf - m) * (xf - m), axis=-1, keepdims=True)
    return (xf - m) * lax.rsqrt(v + 1e-5) * g + b


def _fused_kernel(gath_ref, usm_idx_ref, shop_idx_ref,
                  traits_ref, scalars_ref, sm_tab_ref, shop_tab_ref,
                  tw1_ref, tb1_ref, tw2_ref, tb2_ref, tw3_ref, tb3_ref,
                  sw1_ref, sb1_ref, sw2_ref, sb2_ref, sw3_ref, sb3_ref,
                  base_ref,
                  wq_ref, bq_ref, wk_ref, bk_ref, wv_ref, bv_ref,
                  wo_ref, bo_ref, ln1g_ref, ln1b_ref,
                  w1_ref, b1_ref, w2_ref, b2_ref, ln2g_ref, ln2b_ref,
                  fw1_ref, fb1_ref, fw2_ref, fb2_ref, fw3_ref, fb3_ref,
                  out_ref, x_ref, o_ref):
    f32 = jnp.float32

    g2 = gath_ref[...].reshape(BS, GT, 128)
    usm = _multihot_gather(usm_idx_ref[...], sm_tab_ref[...], SM_ROWS)
    usm = usm.reshape(BS, 40, 128)
    sh = _multihot_gather(shop_idx_ref[...], shop_tab_ref[...], SHOP_ROWS)
    sh = sh.reshape(BS, 8, D)

    t = traits_ref[...]
    t = jnp.maximum(jnp.dot(t, tw1_ref[...], preferred_element_type=f32) + tb1_ref[...], 0.0)
    t = jnp.maximum(jnp.dot(t, tw2_ref[...], preferred_element_type=f32) + tb2_ref[...], 0.0)
    t = jnp.dot(t, tw3_ref[...], preferred_element_type=f32) + tb3_ref[...]
    t = t.reshape(BS, 8, D)

    s = scalars_ref[...]
    s = jnp.maximum(jnp.dot(s, sw1_ref[...], preferred_element_type=f32) + sb1_ref[...], 0.0)
    s = jnp.maximum(jnp.dot(s, sw2_ref[...], preferred_element_type=f32) + sb2_ref[...], 0.0)
    s = jnp.dot(s, sw3_ref[...], preferred_element_type=f32) + sb3_ref[...]
    s = s.reshape(BS, 8, D)

    x_ref[:, 0:4, :] = jnp.zeros((BS, 4, D), f32)
    x_ref[:, 4:32, 0:128] = g2[:, 0:28]
    x_ref[:, 4:32, 128:256] = usm[:, 0:28]
    x_ref[:, 32:39, :] = t[:, 0:7]
    x_ref[:, 39:44, 0:128] = g2[:, 37:42]
    x_ref[:, 39:44, 128:256] = g2[:, 42:47]
    x_ref[:, 44:53, 0:128] = g2[:, 28:37]
    x_ref[:, 44:53, 128:256] = usm[:, 28:37]
    x_ref[:, 53:58, :] = sh[:, 0:5]
    x_ref[:, 58:59, :] = s[:, 0:1]
    x_ref[:, 59:65, 0:128] = g2[:, 47:53]
    x_ref[:, 59:65, 128:256] = g2[:, 53:59]
    x_ref[:, 65:72, :] = jnp.zeros((BS, 7, D), f32)

    xf = (x_ref[...] + base_ref[...][None]).reshape(BS * T, D)

    kmask = jnp.where(
        lax.broadcasted_iota(jnp.int32, (1, 1, T), 2) >= TREAL, -1e30, 0.0)

    for l in range(NL):
        q = jnp.dot(xf, wq_ref[l], preferred_element_type=f32) + bq_ref[l:l + 1]
        k = jnp.dot(xf, wk_ref[l], preferred_element_type=f32) + bk_ref[l:l + 1]
        v = jnp.dot(xf, wv_ref[l], preferred_element_type=f32) + bv_ref[l:l + 1]
        q3 = q.reshape(BS, T, D)
        k3 = k.reshape(BS, T, D)
        v3 = v.reshape(BS, T, D)
        for h in range(H):
            qh = q3[:, :, h * DH:(h + 1) * DH]
            kh = k3[:, :, h * DH:(h + 1) * DH]
            vh = v3[:, :, h * DH:(h + 1) * DH]
            e = jnp.exp(lax.dot_general(
                qh, kh, (((2,), (2,)), ((0,), (0,))),
                preferred_element_type=f32) + kmask)
            rs = lax.reciprocal(jnp.sum(e, axis=-1, keepdims=True))
            o_ref[:, :, h * DH:(h + 1) * DH] = lax.dot_general(
                e, vh, (((2,), (1,)), ((0,), (0,))),
                preferred_element_type=f32) * rs
        o = o_ref[...].reshape(BS * T, D)
        attn = jnp.dot(o, wo_ref[l], preferred_element_type=f32) + bo_ref[l:l + 1]
        xf = _ln(xf + attn, ln1g_ref[l:l + 1], ln1b_ref[l:l + 1])
        h1 = jnp.maximum(jnp.dot(xf, w1_ref[l], preferred_element_type=f32) + b1_ref[l:l + 1], 0.0)
        h2 = jnp.dot(h1, w2_ref[l], preferred_element_type=f32) + b2_ref[l:l + 1]
        xf = _ln(xf + h2, ln2g_ref[l:l + 1], ln2b_ref[l:l + 1])

    x3 = xf.reshape(BS, T, D)
    acc = jnp.zeros((BS, D), f32)
    for tt in range(4):
        acc = acc + jnp.dot(x3[:, tt, :], fw1_ref[tt], preferred_element_type=f32)
    h1 = jnp.maximum(acc + fb1_ref[...], 0.0)
    h2 = jnp.maximum(jnp.dot(h1, fw2_ref[...], preferred_element_type=f32) + fb2_ref[...], 0.0)
    out_ref[...] = jnp.dot(h2, fw3_ref[...], preferred_element_type=f32) + fb3_ref[...]


def kernel(board, bench, shop, items, traits, scalars, emb_scalars, params):
    p = params
    f32 = jnp.float32
    i32 = jnp.int32
    bs = board.shape[0]

    se_stack = jnp.concatenate([
        p['gold_emb'], p['health_emb'], p['exp_emb'],
        p['round_emb'], p['oppo_emb'], p['level_emb']], axis=0)
    t128 = jnp.concatenate([
        p['champion_emb'],
        jnp.zeros((O_IB - 221, 128), f32),
        p['item_bench_emb'],
        jnp.zeros((O_SE - O_IB - 58, 128), f32),
        se_stack.reshape(882, 128),
        jnp.zeros((T128_ROWS - O_Z, 128), f32),
    ], axis=0)

    units = jnp.concatenate(
        [board.reshape(bs, 28, 5), bench.reshape(bs, 9, 5)], axis=1).astype(i32)
    items = items.astype(i32)
    se_v = emb_scalars.astype(i32) + jnp.array(SE_INNER, i32)[None, :]
    gidx = jnp.concatenate([
        units[:, :, 0],
        O_IB + items[:, 0::2],
        O_IB + items[:, 1::2],
        O_SE + 2 * se_v,
        O_SE + 2 * se_v + 1,
        jnp.full((bs, 5), O_Z, i32),
    ], axis=1).reshape(bs * GT)

    gaths = [_sc_gather(t128, gidx[k * _GROWS:(k + 1) * _GROWS])
             for k in range(NSPLIT)]

    sm_tab = jnp.zeros((SM_ROWS, 128), f32)
    sm_tab = sm_tab.at[0:58, 0:32].set(p['item_emb_1'])
    sm_tab = sm_tab.at[64:122, 32:64].set(p['item_emb_2'])
    sm_tab = sm_tab.at[128:186, 64:96].set(p['item_emb_3'])
    sm_tab = sm_tab.at[192:337, 96:128].set(p['champ_trait_emb'])

    shop_tab = jnp.zeros((SHOP_ROWS, D), f32)
    shop_tab = shop_tab.at[0:221, 0:192].set(p['shop_champ_emb'])
    shop_tab = shop_tab.at[224:369, 192:256].set(p['shop_trait_emb'])

    usm_idx = units[:, :, 1:5] + jnp.array(SM_OFF, i32)[None, None, :]
    usm_idx = jnp.pad(usm_idx, ((0, 0), (0, 3), (0, 0))).reshape(bs * 40, 4)

    shop = shop.astype(i32)
    shop_idx = jnp.stack([shop[..., 0], shop[..., 4] + 224], axis=-1)
    shop_idx = jnp.pad(shop_idx, ((0, 0), (0, 3), (0, 0))).reshape(bs * 8, 2)

    traits_p = jnp.pad(traits, ((0, 0), (0, 1), (0, 26))).reshape(bs * 8, 128)
    scalars_p = jnp.pad(scalars, ((0, 0), (0, 7), (0, 4))).reshape(bs * 8, 32)

    tm = p['trait_mlp']
    tw1 = jnp.pad(tm[0][0], ((0, 26), (0, 0)))
    tb1, tw2, tb2, tw3, tb3 = tm[0][1][None], tm[1][0], tm[1][1][None], tm[2][0], tm[2][1][None]
    sm = p['scalar_mlp']
    sw1 = jnp.pad(sm[0][0], ((0, 4), (0, 0)))
    sb1, sw2, sb2, sw3, sb3 = sm[0][1][None], sm[1][0], sm[1][1][None], sm[2][0], sm[2][1][None]
    fp = p['feature_proc']
    fw1 = fp[0][0].reshape(4, D, D)
    fb1, fw2, fb2, fw3, fb3 = fp[0][1][None], fp[1][0], fp[1][1][None], fp[2][0], fp[2][1][None]

    base = jnp.zeros((T, D), f32)
    base = base.at[0:4, :].set(p['cls_token'][0])
    base = base.at[4:65, :].set(p['pos_emb'][0:61])

    L = p['layers']

    def stk(name):
        return jnp.stack([L[l][name] for l in range(NL)])

    iscale = 1.0 / np.sqrt(DH)
    wq, bq, wk, bk = stk('Wq') * iscale, stk('bq') * iscale, stk('Wk'), stk('bk')
    wv, bv, wo, bo = stk('Wv'), stk('bv'), stk('Wo'), stk('bo')
    ln1g, ln1b = stk('ln1_g'), stk('ln1_b')
    w1, b1, w2, b2 = stk('W1'), stk('b1'), stk('W2'), stk('b2')
    ln2g, ln2b = stk('ln2_g'), stk('ln2_b')

    def bspec(shape, blocked_rows=None):
        if blocked_rows is None:
            nd = len(shape)
            return pl.BlockSpec(shape, lambda i: (0,) * nd)
        return pl.BlockSpec((blocked_rows,) + shape[1:],
                            lambda i: (i,) + (0,) * (len(shape) - 1))

    outs = []
    for k in range(NSPLIT):
        lo = k * BSL
        hi = lo + BSL
        ins = [
            (gaths[k], BS * GT),
            (usm_idx[lo * 40:hi * 40], BS * 40),
            (shop_idx[lo * 8:hi * 8], BS * 8),
            (traits_p[lo * 8:hi * 8], BS * 8),
            (scalars_p[lo * 8:hi * 8], BS * 8),
            (sm_tab, None), (shop_tab, None),
            (tw1, None), (tb1, None), (tw2, None), (tb2, None), (tw3, None), (tb3, None),
            (sw1, None), (sb1, None), (sw2, None), (sb2, None), (sw3, None), (sb3, None),
            (base, None),
            (wq, None), (bq, None), (wk, None), (bk, None), (wv, None), (bv, None),
            (wo, None), (bo, None), (ln1g, None), (ln1b, None),
            (w1, None), (b1, None), (w2, None), (b2, None), (ln2g, None), (ln2b, None),
            (fw1, None), (fb1, None), (fw2, None), (fb2, None), (fw3, None), (fb3, None),
        ]
        outs.append(pl.pallas_call(
            _fused_kernel,
            grid=(BSL // BS,),
            in_specs=[bspec(a.shape, r) for a, r in ins],
            out_specs=pl.BlockSpec((BS, 1024), lambda i: (i, 0)),
            out_shape=jax.ShapeDtypeStruct((BSL, 1024), f32),
            scratch_shapes=[pltpu.VMEM((BS, T, D), f32), pltpu.VMEM((BS, T, D), f32)],
            compiler_params=pltpu.CompilerParams(
                dimension_semantics=("arbitrary",)),
        )(*[a for a, _ in ins]))
    return jnp.concatenate(outs, axis=0)


# speedup vs baseline: 1.0354x; 1.0294x over previous
import functools

import jax
import jax.numpy as jnp
import numpy as np
from jax import lax
from jax.experimental import pallas as pl
from jax.experimental.pallas import tpu as pltpu
from jax.experimental.pallas import tpu_sc as plsc

B = 1024
D = 256
H = 8
DH = 32
NL = 4
T = 72
TREAL = 65
BS = 32

GT = 64

O_IB = 224
O_SE = 288
O_Z = 1170
T128_ROWS = 1176
SE_INNER = (0, 61, 162, 263, 303, 431)

SM_ROWS = 384
SM_OFF = (0, 64, 128, 192)
SHOP_ROWS = 384

_NW = 32
NSPLIT = 4
BSL = B // NSPLIT
_GROWS = BSL * GT
_RPW = _GROWS // _NW
_CHUNK = 256
_NCH = _RPW // _CHUNK
_NBUF = min(3, _NCH)


def _sc_gather(table, idx):
    mesh = plsc.VectorSubcoreMesh(core_axis_name="c", subcore_axis_name="s")

    @functools.partial(
        pl.kernel, mesh=mesh,
        out_type=jax.ShapeDtypeStruct((_GROWS, 128), jnp.float32),
        scratch_types=[
            pltpu.VMEM((_RPW,), jnp.int32),
            pltpu.VMEM_SHARED((T128_ROWS, 128), jnp.float32),
        ] + [pltpu.VMEM((_CHUNK, 128), jnp.float32)] * _NBUF + [
            pltpu.SemaphoreType.DMA,
            pltpu.SemaphoreType.DMA,
        ],
    )
    def gather_k(table_hbm, idx_hbm, out_hbm, idx_v, tab_sp, *rest):
        bufs = rest[:_NBUF]
        sem_g, sem_s = rest[_NBUF], rest[_NBUF + 1]
        wid = lax.axis_index("s") * 2 + lax.axis_index("c")
        base = wid * _RPW

        @pl.when(lax.axis_index("s") == 0)
        def _():
            pltpu.sync_copy(table_hbm, tab_sp)
        plsc.subcore_barrier()

        pltpu.sync_copy(idx_hbm.at[pl.ds(base, _RPW)], idx_v)

        def fire(c):
            return pltpu.async_copy(
                tab_sp.at[idx_v.at[pl.ds(c * _CHUNK, _CHUNK)]],
                bufs[c % _NBUF], sem_g)

        gets = {c: fire(c) for c in range(_NBUF)}
        puts = {}
        for c in range(_NCH):
            gets[c].wait()
            puts[c] = pltpu.async_copy(
                bufs[c % _NBUF], out_hbm.at[pl.ds(base + c * _CHUNK, _CHUNK)],
                sem_s)
            if c + _NBUF < _NCH:
                puts[c].wait()
                gets[c + _NBUF] = fire(c + _NBUF)
        for c in range(max(0, _NCH - _NBUF), _NCH):
            puts[c].wait()

    return gather_k(table, idx)


def _multihot_gather(idx, tab, nrows):
    rows = idx.shape[0]
    iota = lax.broadcasted_iota(jnp.int32, (rows, nrows), 1)
    mh = jnp.zeros((rows, nrows), jnp.float32)
    for f in range(idx.shape[1]):
        mh = mh + (iota == idx[:, f:f + 1]).astype(jnp.float32)
    return jnp.dot(mh, tab, preferred_element_type=jnp.float32)


def _ln(xf, g, b):
    m = jnp.mean(xf, axis=-1, keepdims=True)
    v = jnp.mean((xf - m) * (xf - m), axis=-1, keepdims=True)
    return (xf - m) * lax.rsqrt(v + 1e-5) * g + b


def _fused_kernel(gath_ref, usm_idx_ref, shop_idx_ref,
                  traits_ref, scalars_ref, sm_tab_ref, shop_tab_ref,
                  tw1_ref, tb1_ref, tw2_ref, tb2_ref, tw3_ref, tb3_ref,
                  sw1_ref, sb1_ref, sw2_ref, sb2_ref, sw3_ref, sb3_ref,
                  base_ref,
                  wq_ref, bq_ref, wk_ref, bk_ref, wv_ref, bv_ref,
                  wo_ref, bo_ref, ln1g_ref, ln1b_ref,
                  w1_ref, b1_ref, w2_ref, b2_ref, ln2g_ref, ln2b_ref,
                  fw1_ref, fb1_ref, fw2_ref, fb2_ref, fw3_ref, fb3_ref,
                  out_ref, x_ref, o_ref):
    f32 = jnp.float32

    g2 = gath_ref[...].reshape(BS, GT, 128)
    usm = _multihot_gather(usm_idx_ref[...], sm_tab_ref[...], SM_ROWS)
    usm = usm.reshape(BS, 40, 128)
    sh = _multihot_gather(shop_idx_ref[...], shop_tab_ref[...], SHOP_ROWS)
    sh = sh.reshape(BS, 8, D)

    t = traits_ref[...]
    t = jnp.maximum(jnp.dot(t, tw1_ref[...], preferred_element_type=f32) + tb1_ref[...], 0.0)
    t = jnp.maximum(jnp.dot(t, tw2_ref[...], preferred_element_type=f32) + tb2_ref[...], 0.0)
    t = jnp.dot(t, tw3_ref[...], preferred_element_type=f32) + tb3_ref[...]
    t = t.reshape(BS, 8, D)

    s = scalars_ref[...]
    s = jnp.maximum(jnp.dot(s, sw1_ref[...], preferred_element_type=f32) + sb1_ref[...], 0.0)
    s = jnp.maximum(jnp.dot(s, sw2_ref[...], preferred_element_type=f32) + sb2_ref[...], 0.0)
    s = jnp.dot(s, sw3_ref[...], preferred_element_type=f32) + sb3_ref[...]
    s = s.reshape(BS, 8, D)

    x_ref[:, 0:4, :] = jnp.zeros((BS, 4, D), f32)
    x_ref[:, 4:32, 0:128] = g2[:, 0:28]
    x_ref[:, 4:32, 128:256] = usm[:, 0:28]
    x_ref[:, 32:39, :] = t[:, 0:7]
    x_ref[:, 39:44, 0:128] = g2[:, 37:42]
    x_ref[:, 39:44, 128:256] = g2[:, 42:47]
    x_ref[:, 44:53, 0:128] = g2[:, 28:37]
    x_ref[:, 44:53, 128:256] = usm[:, 28:37]
    x_ref[:, 53:58, :] = sh[:, 0:5]
    x_ref[:, 58:59, :] = s[:, 0:1]
    x_ref[:, 59:65, 0:128] = g2[:, 47:53]
    x_ref[:, 59:65, 128:256] = g2[:, 53:59]
    x_ref[:, 65:72, :] = jnp.zeros((BS, 7, D), f32)

    xf = (x_ref[...] + base_ref[...][None]).reshape(BS * T, D)

    kmask = jnp.where(
        lax.broadcasted_iota(jnp.int32, (1, 1, T), 2) >= TREAL, -1e30, 0.0)

    for l in range(NL):
        q = jnp.dot(xf, wq_ref[l], preferred_element_type=f32) + bq_ref[l:l + 1]
        k = jnp.dot(xf, wk_ref[l], preferred_element_type=f32) + bk_ref[l:l + 1]
        v = jnp.dot(xf, wv_ref[l], preferred_element_type=f32) + bv_ref[l:l + 1]
        q3 = q.reshape(BS, T, D)
        k3 = k.reshape(BS, T, D)
        v3 = v.reshape(BS, T, D)
        for h in range(H):
            qh = q3[:, :, h * DH:(h + 1) * DH]
            kh = k3[:, :, h * DH:(h + 1) * DH]
            vh = v3[:, :, h * DH:(h + 1) * DH]
            e = jnp.exp(lax.dot_general(
                qh, kh, (((2,), (2,)), ((0,), (0,))),
                preferred_element_type=f32) + kmask)
            rs = lax.reciprocal(jnp.sum(e, axis=-1, keepdims=True))
            o_ref[:, :, h * DH:(h + 1) * DH] = lax.dot_general(
                e, vh, (((2,), (1,)), ((0,), (0,))),
                preferred_element_type=f32) * rs
        o = o_ref[...].reshape(BS * T, D)
        attn = jnp.dot(o, wo_ref[l], preferred_element_type=f32) + bo_ref[l:l + 1]
        xf = _ln(xf + attn, ln1g_ref[l:l + 1], ln1b_ref[l:l + 1])
        h1 = jnp.maximum(jnp.dot(xf, w1_ref[l], preferred_element_type=f32) + b1_ref[l:l + 1], 0.0)
        h2 = jnp.dot(h1, w2_ref[l], preferred_element_type=f32) + b2_ref[l:l + 1]
        xf = _ln(xf + h2, ln2g_ref[l:l + 1], ln2b_ref[l:l + 1])

    x3 = xf.reshape(BS, T, D)
    acc = jnp.zeros((BS, D), f32)
    for tt in range(4):
        acc = acc + jnp.dot(x3[:, tt, :], fw1_ref[tt], preferred_element_type=f32)
    h1 = jnp.maximum(acc + fb1_ref[...], 0.0)
    h2 = jnp.maximum(jnp.dot(h1, fw2_ref[...], preferred_element_type=f32) + fb2_ref[...], 0.0)
    out_ref[...] = jnp.dot(h2, fw3_ref[...], preferred_element_type=f32) + fb3_ref[...]


def kernel(board, bench, shop, items, traits, scalars, emb_scalars, params):
    p = params
    f32 = jnp.float32
    i32 = jnp.int32
    bs = board.shape[0]

    se_stack = jnp.concatenate([
        p['gold_emb'], p['health_emb'], p['exp_emb'],
        p['round_emb'], p['oppo_emb'], p['level_emb']], axis=0)
    t128 = jnp.concatenate([
        p['champion_emb'],
        jnp.zeros((O_IB - 221, 128), f32),
        p['item_bench_emb'],
        jnp.zeros((O_SE - O_IB - 58, 128), f32),
        se_stack.reshape(882, 128),
        jnp.zeros((T128_ROWS - O_Z, 128), f32),
    ], axis=0)

    units = jnp.concatenate(
        [board.reshape(bs, 28, 5), bench.reshape(bs, 9, 5)], axis=1).astype(i32)
    items = items.astype(i32)
    se_v = emb_scalars.astype(i32) + jnp.array(SE_INNER, i32)[None, :]
    gidx = jnp.concatenate([
        units[:, :, 0],
        O_IB + items[:, 0::2],
        O_IB + items[:, 1::2],
        O_SE + 2 * se_v,
        O_SE + 2 * se_v + 1,
        jnp.full((bs, 5), O_Z, i32),
    ], axis=1).reshape(bs * GT)

    gaths = [_sc_gather(t128, gidx[k * _GROWS:(k + 1) * _GROWS])
             for k in range(NSPLIT)]

    sm_tab = jnp.zeros((SM_ROWS, 128), f32)
    sm_tab = sm_tab.at[0:58, 0:32].set(p['item_emb_1'])
    sm_tab = sm_tab.at[64:122, 32:64].set(p['item_emb_2'])
    sm_tab = sm_tab.at[128:186, 64:96].set(p['item_emb_3'])
    sm_tab = sm_tab.at[192:337, 96:128].set(p['champ_trait_emb'])

    shop_tab = jnp.zeros((SHOP_ROWS, D), f32)
    shop_tab = shop_tab.at[0:221, 0:192].set(p['shop_champ_emb'])
    shop_tab = shop_tab.at[224:369, 192:256].set(p['shop_trait_emb'])

    usm_idx = units[:, :, 1:5] + jnp.array(SM_OFF, i32)[None, None, :]
    usm_idx = jnp.pad(usm_idx, ((0, 0), (0, 3), (0, 0))).reshape(bs * 40, 4)

    shop = shop.astype(i32)
    shop_idx = jnp.stack([shop[..., 0], shop[..., 4] + 224], axis=-1)
    shop_idx = jnp.pad(shop_idx, ((0, 0), (0, 3), (0, 0))).reshape(bs * 8, 2)

    traits_p = jnp.pad(traits, ((0, 0), (0, 1), (0, 26))).reshape(bs * 8, 128)
    scalars_p = jnp.pad(scalars, ((0, 0), (0, 7), (0, 4))).reshape(bs * 8, 32)

    tm = p['trait_mlp']
    tw1 = jnp.pad(tm[0][0], ((0, 26), (0, 0)))
    tb1, tw2, tb2, tw3, tb3 = tm[0][1][None], tm[1][0], tm[1][1][None], tm[2][0], tm[2][1][None]
    sm = p['scalar_mlp']
    sw1 = jnp.pad(sm[0][0], ((0, 4), (0, 0)))
    sb1, sw2, sb2, sw3, sb3 = sm[0][1][None], sm[1][0], sm[1][1][None], sm[2][0], sm[2][1][None]
    fp = p['feature_proc']
    fw1 = fp[0][0].reshape(4, D, D)
    fb1, fw2, fb2, fw3, fb3 = fp[0][1][None], fp[1][0], fp[1][1][None], fp[2][0], fp[2][1][None]

    base = jnp.zeros((T, D), f32)
    base = base.at[0:4, :].set(p['cls_token'][0])
    base = base.at[4:65, :].set(p['pos_emb'][0:61])

    L = p['layers']

    def stk(name):
        return jnp.stack([L[l][name] for l in range(NL)])

    iscale = 1.0 / np.sqrt(DH)
    wq, bq, wk, bk = stk('Wq') * iscale, stk('bq') * iscale, stk('Wk'), stk('bk')
    wv, bv, wo, bo = stk('Wv'), stk('bv'), stk('Wo'), stk('bo')
    ln1g, ln1b = stk('ln1_g'), stk('ln1_b')
    w1, b1, w2, b2 = stk('W1'), stk('b1'), stk('W2'), stk('b2')
    ln2g, ln2b = stk('ln2_g'), stk('ln2_b')

    def bspec(shape, blocked_rows=None):
        if blocked_rows is None:
            nd = len(shape)
            return pl.BlockSpec(shape, lambda i: (0,) * nd)
        return pl.BlockSpec((blocked_rows,) + shape[1:],
                            lambda i: (i,) + (0,) * (len(shape) - 1))

    outs = []
    for k in range(NSPLIT):
        lo = k * BSL
        hi = lo + BSL
        ins = [
            (gaths[k], BS * GT),
            (usm_idx[lo * 40:hi * 40], BS * 40),
            (shop_idx[lo * 8:hi * 8], BS * 8),
            (traits_p[lo * 8:hi * 8], BS * 8),
            (scalars_p[lo * 8:hi * 8], BS * 8),
            (sm_tab, None), (shop_tab, None),
            (tw1, None), (tb1, None), (tw2, None), (tb2, None), (tw3, None), (tb3, None),
            (sw1, None), (sb1, None), (sw2, None), (sb2, None), (sw3, None), (sb3, None),
            (base, None),
            (wq, None), (bq, None), (wk, None), (bk, None), (wv, None), (bv, None),
            (wo, None), (bo, None), (ln1g, None), (ln1b, None),
            (w1, None), (b1, None), (w2, None), (b2, None), (ln2g, None), (ln2b, None),
            (fw1, None), (fb1, None), (fw2, None), (fb2, None), (fw3, None), (fb3, None),
        ]
        outs.append(pl.pallas_call(
            _fused_kernel,
            grid=(BSL // BS,),
            in_specs=[bspec(a.shape, r) for a, r in ins],
            out_specs=pl.BlockSpec((BS, 1024), lambda i: (i, 0)),
            out_shape=jax.ShapeDtypeStruct((BSL, 1024), f32),
            scratch_shapes=[pltpu.VMEM((BS, T, D), f32), pltpu.VMEM((BS, T, D), f32)],
            compiler_params=pltpu.CompilerParams(
                dimension_semantics=("arbitrary",)),
        )(*[a for a, _ in ins]))
    return jnp.concatenate(outs, axis=0)
